# baseline (speedup 1.0000x reference)
import functools

import jax
import jax.numpy as jnp
from jax import lax
from jax.experimental import pallas as pl
from jax.experimental.pallas import tpu as pltpu
from jax.experimental.pallas import tpu_sc as plsc

NC = 2
NS = 16
NT = NC * NS
KE = 128
LANES = 16


def _relval_body(rel_ref, w_ref, b_ref, out_ref):
    out_ref[...] = lax.dot_general(
        rel_ref[...], w_ref[...], (((1,), (1,)), ((), ())),
        preferred_element_type=jnp.float32) + b_ref[...]


def _addrelu_body(p_ref, o_ref):
    o_ref[...] = jnp.maximum(p_ref[0] + p_ref[1], 0.0)


@functools.lru_cache(maxsize=None)
def _make_sc_scatter(NB, NENT, H):
    assert NB % 2 == 0
    chunk = 200
    nchunk = NENT // chunk
    cbase, cextra = nchunk // NS, nchunk % NS
    mesh = plsc.VectorSubcoreMesh(core_axis_name="c", subcore_axis_name="s")

    @functools.partial(
        pl.kernel,
        out_type=jax.ShapeDtypeStruct((NC, NENT, H), jnp.float32),
        mesh=mesh,
        scratch_types=[
            pltpu.VMEM_SHARED((NENT, H), jnp.float32),
            pltpu.VMEM((4, KE), jnp.int32),
            pltpu.VMEM((4, KE), jnp.int32),
            pltpu.VMEM((KE, H), jnp.float32),
            pltpu.VMEM((KE, H), jnp.float32),
            pltpu.SemaphoreType.DMA,
            pltpu.SemaphoreType.DMA,
            pltpu.SemaphoreType.DMA,
            pltpu.SemaphoreType.DMA,
        ],
    )
    def sc_scatter(relval_hbm, slab_hbm, zeros_hbm, out_hbm, acc,
                   sl0, sl1, rows0, rows1, ss0, ss1, gs0, gs1):
        cid = lax.axis_index("c")
        sid = lax.axis_index("s")
        wid = sid * NC + cid

        nch = cbase + jnp.where(sid < cextra, 1, 0)

        def zero_body(j, carry):
            c = sid + NS * j
            pltpu.sync_copy(zeros_hbm, acc.at[pl.ds(c * chunk, chunk)])
            return carry

        lax.fori_loop(0, nch, zero_body, 0)

        def slab_start(j, sl, sem):
            pltpu.async_copy(slab_hbm.at[wid, j], sl, sem)

        def slab_wait(j, sl, sem):
            pltpu.make_async_copy(slab_hbm.at[wid, j], sl, sem).wait()

        def gather_start(sl, rows, sem):
            pltpu.async_copy(relval_hbm.at[sl.at[0]], rows, sem)

        def gather_wait(sl, rows, sem):
            pltpu.make_async_copy(relval_hbm.at[sl.at[0]], rows, sem).wait()

        def process(sl, rows, gsem):
            gather_wait(sl, rows, gsem)

            def scale(grp, c2):
                wvec = lax.bitcast_convert_type(
                    sl[3, pl.ds(grp * LANES, LANES)], jnp.float32)
                for i in range(LANES):
                    k = grp * LANES + i
                    wk = wvec[i]
                    for c in range(H // LANES):
                        s = pl.ds(c * LANES, LANES)
                        rows[k, s] = rows[k, s] * wk
                return c2

            lax.fori_loop(0, KE // LANES, scale, 0)
            pltpu.sync_copy(rows, acc.at[sl.at[1]], add=True)
            pltpu.sync_copy(rows, acc.at[sl.at[2]], add=True)

        slab_start(0, sl0, ss0)
        slab_wait(0, sl0, ss0)
        gather_start(sl0, rows0, gs0)
        slab_start(1, sl1, ss1)
        plsc.subcore_barrier()

        def pair_body(jj, carry):
            j0 = 2 * jj
            slab_wait(j0 + 1, sl1, ss1)
            gather_start(sl1, rows1, gs1)
            process(sl0, rows0, gs0)
            slab_start(jnp.minimum(j0 + 2, NB - 1), sl0, ss0)
            slab_wait(j0 + 2, sl0, ss0)
            gather_start(sl0, rows0, gs0)
            process(sl1, rows1, gs1)
            slab_start(jnp.minimum(j0 + 3, NB - 1), sl1, ss1)
            return carry

        lax.fori_loop(0, NB // 2, pair_body, 0)
        gather_wait(sl0, rows0, gs0)
        slab_wait(NB - 1, sl1, ss1)
        plsc.subcore_barrier()

        def out_body(j, carry):
            c = sid + NS * j
            s = pl.ds(c * chunk, chunk)
            pltpu.sync_copy(acc.at[s], out_hbm.at[cid, s])
            return carry

        lax.fori_loop(0, nch, out_body, 0)

    return sc_scatter


def kernel(local_entity, batch_heads, batch_rels, batch_tails, batch_ids,
           fact_ids, weight_list, weight_rel_list, rel_features, W, b):
    bsz, max_local_entity = local_entity.shape
    nent = bsz * max_local_entity
    R, H = rel_features.shape
    E = batch_rels.shape[0]

    rel_val = pl.pallas_call(
        _relval_body,
        out_shape=jax.ShapeDtypeStruct((R, H), jnp.float32),
    )(rel_features, W, b.reshape(1, H))

    nb = -(-E // (NT * KE))
    nb += nb % 2
    ep = NT * nb * KE - E

    def prep(x):
        return jnp.pad(x.astype(jnp.int32), (0, ep)).reshape(NT, nb, 1, KE)

    slab = jnp.concatenate([
        prep(batch_rels),
        prep(batch_tails),
        prep(batch_heads),
        prep(lax.bitcast_convert_type(
            weight_rel_list.astype(jnp.float32), jnp.int32)),
    ], axis=2)
    zeros = jnp.zeros((200, H), jnp.float32)

    part = _make_sc_scatter(nb, nent, H)(rel_val, slab, zeros)

    rows_blk = 2000
    out = pl.pallas_call(
        _addrelu_body,
        grid=(nent // rows_blk,),
        in_specs=[pl.BlockSpec((NC, rows_blk, H), lambda i: (0, i, 0))],
        out_specs=pl.BlockSpec((rows_blk, H), lambda i: (i, 0)),
        out_shape=jax.ShapeDtypeStruct((nent, H), jnp.float32),
    )(part)
    return out.reshape(bsz, max_local_entity, H)


# speedup vs baseline: 1.4879x; 1.4879x over previous
import functools

import jax
import jax.numpy as jnp
from jax import lax
from jax.experimental import pallas as pl
from jax.experimental.pallas import tpu as pltpu
from jax.experimental.pallas import tpu_sc as plsc

NC = 2
NS = 16
NT = NC * NS
KE = 112
LANES = 16
SRING = 6
RRING = 3


def _relval_body(rel_ref, w_ref, b_ref, out_ref):
    out_ref[...] = lax.dot_general(
        rel_ref[...], w_ref[...], (((1,), (1,)), ((), ())),
        preferred_element_type=jnp.float32) + b_ref[...]


def _addrelu_body(p_ref, o_ref):
    o_ref[...] = jnp.maximum(p_ref[0] + p_ref[1], 0.0)


@functools.lru_cache(maxsize=None)
def _make_sc_scatter(NB, NENT, R, H):
    assert NB % SRING == 0 and NB >= 2 * SRING
    zch = 80
    nzch = NENT // zch
    zbase, zextra = nzch // NS, nzch % NS
    chunk = 200
    nchunk = NENT // chunk
    cbase, cextra = nchunk // NS, nchunk % NS
    mesh = plsc.VectorSubcoreMesh(core_axis_name="c", subcore_axis_name="s")

    @functools.partial(
        pl.kernel,
        out_type=jax.ShapeDtypeStruct((NC, NENT, H), jnp.float32),
        mesh=mesh,
        scratch_types=(
            [pltpu.VMEM_SHARED((NENT, H), jnp.float32)]
            + [pltpu.VMEM((4, KE), jnp.int32)] * SRING
            + [pltpu.VMEM((KE, H), jnp.float32)] * RRING
            + [pltpu.SemaphoreType.DMA] * (SRING + 3 * RRING)
        ),
    )
    def sc_scatter(relval_hbm, slab_hbm, out_hbm, acc, *bufs):
        slabs = bufs[0:SRING]
        rows = bufs[SRING:SRING + RRING]
        ssem = bufs[SRING + RRING:2 * SRING + RRING]
        gsem = bufs[2 * SRING + RRING:2 * SRING + 2 * RRING]
        tsem = bufs[2 * SRING + 2 * RRING:2 * SRING + 3 * RRING]
        hsem = bufs[2 * SRING + 3 * RRING:2 * SRING + 4 * RRING]
        cid = lax.axis_index("c")
        sid = lax.axis_index("s")
        wid = sid * NC + cid

        def zfill(r, carry):
            for c in range(H // LANES):
                rows[0][r, pl.ds(c * LANES, LANES)] = jnp.zeros(
                    (LANES,), jnp.float32)
            return carry

        lax.fori_loop(0, zch, zfill, 0)
        nch0 = zbase + jnp.where(sid < zextra, 1, 0)

        def zero_body(j, carry):
            c = sid + NS * j
            pltpu.sync_copy(rows[0].at[pl.ds(0, zch)],
                            acc.at[pl.ds(c * zch, zch)])
            return carry

        lax.fori_loop(0, nch0, zero_body, 0)

        def slab_start(j, s):
            pltpu.async_copy(slab_hbm.at[wid, j], slabs[s], ssem[s])

        def slab_wait(j, s):
            pltpu.make_async_copy(slab_hbm.at[wid, j], slabs[s],
                                  ssem[s]).wait()

        def gather_start(s, r):
            pltpu.async_copy(relval_hbm.at[slabs[s].at[0]], rows[r], gsem[r])

        def gather_wait(s, r):
            pltpu.make_async_copy(relval_hbm.at[slabs[s].at[0]], rows[r],
                                  gsem[r]).wait()

        def scat_start(s, r):
            pltpu.async_copy(rows[r], acc.at[slabs[s].at[1]], tsem[r],
                             add=True)
            pltpu.async_copy(rows[r], acc.at[slabs[s].at[2]], hsem[r],
                             add=True)

        def scat_wait(s, r):
            pltpu.make_async_copy(rows[r], acc.at[slabs[s].at[1]],
                                  tsem[r]).wait()
            pltpu.make_async_copy(rows[r], acc.at[slabs[s].at[2]],
                                  hsem[r]).wait()

        def scale(s, r):
            sl, rw = slabs[s], rows[r]

            def grp_body(grp, c2):
                wvec = lax.bitcast_convert_type(
                    sl[3, pl.ds(grp * LANES, LANES)], jnp.float32)
                for i in range(LANES):
                    k = grp * LANES + i
                    wk = wvec[i]
                    for c in range(H // LANES):
                        cs = pl.ds(c * LANES, LANES)
                        rw[k, cs] = rw[k, cs] * wk
                return c2

            lax.fori_loop(0, KE // LANES, grp_body, 0)

        plsc.subcore_barrier()
        for s in range(4):
            slab_start(s, s)
        slab_wait(0, 0)
        gather_start(0, 0)

        def ring_body(g, carry):
            for k in range(SRING):
                j = SRING * g + k

                def step(jv):
                    scat_wait((k + 4) % SRING, (k + 1) % RRING)

                if k < 2:
                    @pl.when(g >= 1)
                    def _():
                        step(j)
                else:
                    step(j)
                slab_start(jnp.minimum(j + 4, NB - 1), (k + 4) % SRING)
                slab_wait(jnp.minimum(j + 1, NB - 1), (k + 1) % SRING)
                gather_start((k + 1) % SRING, (k + 1) % RRING)
                gather_wait(k % SRING, k % RRING)
                scale(k % SRING, k % RRING)
                scat_start(k % SRING, k % RRING)
            return carry

        lax.fori_loop(0, NB // SRING, ring_body, 0)
        scat_wait((NB - 2) % SRING, (NB - 2) % RRING)
        scat_wait((NB - 1) % SRING, (NB - 1) % RRING)
        gather_wait(NB % SRING, NB % RRING)
        for d in range(1, 4):
            slab_wait(NB - 1, (NB + d) % SRING)
        plsc.subcore_barrier()

        nch = cbase + jnp.where(sid < cextra, 1, 0)

        def out_body(j, carry):
            c = sid + NS * j
            s = pl.ds(c * chunk, chunk)
            pltpu.sync_copy(acc.at[s], out_hbm.at[cid, s])
            return carry

        lax.fori_loop(0, nch, out_body, 0)

    return sc_scatter


def kernel(local_entity, batch_heads, batch_rels, batch_tails, batch_ids,
           fact_ids, weight_list, weight_rel_list, rel_features, W, b):
    bsz, max_local_entity = local_entity.shape
    nent = bsz * max_local_entity
    R, H = rel_features.shape
    E = batch_rels.shape[0]

    rel_val = pl.pallas_call(
        _relval_body,
        out_shape=jax.ShapeDtypeStruct((R, H), jnp.float32),
    )(rel_features, W, b.reshape(1, H))

    nb = -(-E // (NT * KE))
    nb += (-nb) % SRING
    nb = max(nb, 2 * SRING)
    ep = NT * nb * KE - E

    def prep(x):
        return jnp.pad(x.astype(jnp.int32), (0, ep)).reshape(NT, nb, 1, KE)

    slab = jnp.concatenate([
        prep(batch_rels),
        prep(batch_tails),
        prep(batch_heads),
        prep(lax.bitcast_convert_type(
            weight_rel_list.astype(jnp.float32), jnp.int32)),
    ], axis=2)

    part = _make_sc_scatter(nb, nent, R, H)(rel_val, slab)

    rows_blk = 2000
    out = pl.pallas_call(
        _addrelu_body,
        grid=(nent // rows_blk,),
        in_specs=[pl.BlockSpec((NC, rows_blk, H), lambda i: (0, i, 0))],
        out_specs=pl.BlockSpec((rows_blk, H), lambda i: (i, 0)),
        out_shape=jax.ShapeDtypeStruct((nent, H), jnp.float32),
    )(part)
    return out.reshape(bsz, max_local_entity, H)


# speedup vs baseline: 1.7250x; 1.1594x over previous
import functools

import jax
import jax.numpy as jnp
from jax import lax
from jax.experimental import pallas as pl
from jax.experimental.pallas import tpu as pltpu
from jax.experimental.pallas import tpu_sc as plsc

NC = 2
NS = 16
NT = NC * NS
KE = 112
LANES = 16
SRING = 6
RRING = 3


def _relval_body(rel_ref, w_ref, b_ref, out_ref):
    out_ref[...] = lax.dot_general(
        rel_ref[...], w_ref[...], (((1,), (1,)), ((), ())),
        preferred_element_type=jnp.float32) + b_ref[...]


def _addrelu_body(p_ref, o_ref):
    o_ref[...] = jnp.maximum(p_ref[0] + p_ref[1], 0.0)


@functools.lru_cache(maxsize=None)
def _make_sc_scatter(NB, NENT, R, H):
    assert NB % SRING == 0 and NB >= 2 * SRING
    zch = 80
    nzch = NENT // zch
    zbase, zextra = nzch // NS, nzch % NS
    chunk = 200
    nchunk = NENT // chunk
    cbase, cextra = nchunk // NS, nchunk % NS
    mesh = plsc.VectorSubcoreMesh(core_axis_name="c", subcore_axis_name="s")

    @functools.partial(
        pl.kernel,
        out_type=jax.ShapeDtypeStruct((NC, NENT, H), jnp.float32),
        mesh=mesh,
        scratch_types=(
            [pltpu.VMEM_SHARED((NENT, H), jnp.float32)]
            + [pltpu.VMEM((4, KE), jnp.int32)] * SRING
            + [pltpu.VMEM((KE, H), jnp.float32)] * RRING
            + [pltpu.SemaphoreType.DMA] * (SRING + 3 * RRING)
        ),
    )
    def sc_scatter(relval_hbm, rels_hbm, tails_hbm, heads_hbm, w_hbm,
                   out_hbm, acc, *bufs):
        edge_hbm = (rels_hbm, tails_hbm, heads_hbm, w_hbm)
        slabs = bufs[0:SRING]
        rows = bufs[SRING:SRING + RRING]
        ssem = bufs[SRING + RRING:2 * SRING + RRING]
        gsem = bufs[2 * SRING + RRING:2 * SRING + 2 * RRING]
        tsem = bufs[2 * SRING + 2 * RRING:2 * SRING + 3 * RRING]
        hsem = bufs[2 * SRING + 3 * RRING:2 * SRING + 4 * RRING]
        cid = lax.axis_index("c")
        sid = lax.axis_index("s")
        wid = sid * NC + cid

        def zfill(r, carry):
            for c in range(H // LANES):
                rows[0][r, pl.ds(c * LANES, LANES)] = jnp.zeros(
                    (LANES,), jnp.float32)
            return carry

        lax.fori_loop(0, zch, zfill, 0)
        nch0 = zbase + jnp.where(sid < zextra, 1, 0)

        def zero_body(j, carry):
            c = sid + NS * j
            pltpu.sync_copy(rows[0].at[pl.ds(0, zch)],
                            acc.at[pl.ds(c * zch, zch)])
            return carry

        lax.fori_loop(0, nch0, zero_body, 0)

        def slab_start(j, s):
            eoff = (wid * NB + j) * KE
            for a in range(4):
                pltpu.async_copy(edge_hbm[a].at[pl.ds(eoff, KE)],
                                 slabs[s].at[a], ssem[s])

        def slab_wait(j, s):
            eoff = (wid * NB + j) * KE
            for a in range(4):
                pltpu.make_async_copy(edge_hbm[a].at[pl.ds(eoff, KE)],
                                      slabs[s].at[a], ssem[s]).wait()

        def gather_start(s, r):
            pltpu.async_copy(relval_hbm.at[slabs[s].at[0]], rows[r], gsem[r])

        def gather_wait(s, r):
            pltpu.make_async_copy(relval_hbm.at[slabs[s].at[0]], rows[r],
                                  gsem[r]).wait()

        def scat_start(s, r):
            pltpu.async_copy(rows[r], acc.at[slabs[s].at[1]], tsem[r],
                             add=True)
            pltpu.async_copy(rows[r], acc.at[slabs[s].at[2]], hsem[r],
                             add=True)

        def scat_wait(s, r):
            pltpu.make_async_copy(rows[r], acc.at[slabs[s].at[1]],
                                  tsem[r]).wait()
            pltpu.make_async_copy(rows[r], acc.at[slabs[s].at[2]],
                                  hsem[r]).wait()

        def scale(s, r):
            sl, rw = slabs[s], rows[r]

            def grp_body(grp, c2):
                wvec = lax.bitcast_convert_type(
                    sl[3, pl.ds(grp * LANES, LANES)], jnp.float32)
                for i in range(LANES):
                    k = grp * LANES + i
                    wk = wvec[i]
                    for c in range(H // LANES):
                        cs = pl.ds(c * LANES, LANES)
                        rw[k, cs] = rw[k, cs] * wk
                return c2

            lax.fori_loop(0, KE // LANES, grp_body, 0)

        plsc.subcore_barrier()
        for s in range(4):
            slab_start(s, s)
        slab_wait(0, 0)
        gather_start(0, 0)

        def ring_body(g, carry):
            for k in range(SRING):
                j = SRING * g + k

                def step(jv):
                    scat_wait((k + 4) % SRING, (k + 1) % RRING)

                if k < 2:
                    @pl.when(g >= 1)
                    def _():
                        step(j)
                else:
                    step(j)
                slab_start(jnp.minimum(j + 4, NB - 1), (k + 4) % SRING)
                slab_wait(jnp.minimum(j + 1, NB - 1), (k + 1) % SRING)
                gather_start((k + 1) % SRING, (k + 1) % RRING)
                gather_wait(k % SRING, k % RRING)
                scale(k % SRING, k % RRING)
                scat_start(k % SRING, k % RRING)
            return carry

        lax.fori_loop(0, NB // SRING, ring_body, 0)
        scat_wait((NB - 2) % SRING, (NB - 2) % RRING)
        scat_wait((NB - 1) % SRING, (NB - 1) % RRING)
        gather_wait(NB % SRING, NB % RRING)
        for d in range(1, 4):
            slab_wait(NB - 1, (NB + d) % SRING)
        plsc.subcore_barrier()

        nch = cbase + jnp.where(sid < cextra, 1, 0)

        def out_body(j, carry):
            c = sid + NS * j
            s = pl.ds(c * chunk, chunk)
            pltpu.sync_copy(acc.at[s], out_hbm.at[cid, s])
            return carry

        lax.fori_loop(0, nch, out_body, 0)

    return sc_scatter


def kernel(local_entity, batch_heads, batch_rels, batch_tails, batch_ids,
           fact_ids, weight_list, weight_rel_list, rel_features, W, b):
    bsz, max_local_entity = local_entity.shape
    nent = bsz * max_local_entity
    R, H = rel_features.shape
    E = batch_rels.shape[0]

    rel_val = pl.pallas_call(
        _relval_body,
        out_shape=jax.ShapeDtypeStruct((R, H), jnp.float32),
    )(rel_features, W, b.reshape(1, H))

    nb = -(-E // (NT * KE))
    nb += (-nb) % SRING
    nb = max(nb, 2 * SRING)
    ep = NT * nb * KE - E

    def prep(x):
        return jnp.pad(x.astype(jnp.int32), (0, ep))

    part = _make_sc_scatter(nb, nent, R, H)(
        rel_val,
        prep(batch_rels),
        prep(batch_tails),
        prep(batch_heads),
        prep(lax.bitcast_convert_type(
            weight_rel_list.astype(jnp.float32), jnp.int32)),
    )

    rows_blk = 2000
    out = pl.pallas_call(
        _addrelu_body,
        grid=(nent // rows_blk,),
        in_specs=[pl.BlockSpec((NC, rows_blk, H), lambda i: (0, i, 0))],
        out_specs=pl.BlockSpec((rows_blk, H), lambda i: (i, 0)),
        out_shape=jax.ShapeDtypeStruct((nent, H), jnp.float32),
    )(part)
    return out.reshape(bsz, max_local_entity, H)


# speedup vs baseline: 2.1447x; 1.2433x over previous
import functools

import jax
import jax.numpy as jnp
from jax import lax
from jax.experimental import pallas as pl
from jax.experimental.pallas import tpu as pltpu
from jax.experimental.pallas import tpu_sc as plsc

NC = 2
NS = 16
NT = NC * NS
KE = 112
LANES = 16
SRING = 6
RRING = 2


def _relval_body(rel_ref, w_ref, b_ref, out_ref):
    out_ref[...] = lax.dot_general(
        rel_ref[...], w_ref[...], (((1,), (1,)), ((), ())),
        preferred_element_type=jnp.float32) + b_ref[...]


def _addrelu_body(p_ref, o_ref):
    o_ref[...] = jnp.maximum(p_ref[0] + p_ref[1], 0.0)


@functools.lru_cache(maxsize=None)
def _make_sc_scatter(NB0, NB1, NENT, R, H):
    assert NB0 % SRING == 0 and NB0 >= 2 * SRING
    assert NB1 % SRING == 0 and NB1 >= 2 * SRING
    zch = 80
    nzch = NENT // zch
    zbase, zextra = nzch // NS, nzch % NS
    chunk = 200
    nchunk = NENT // chunk
    cbase, cextra = nchunk // NS, nchunk % NS
    rch = 40
    assert R % rch == 0
    nrch = R // rch
    rbase, rextra = nrch // NS, nrch % NS
    mesh = plsc.VectorSubcoreMesh(core_axis_name="c", subcore_axis_name="s")

    @functools.partial(
        pl.kernel,
        out_type=jax.ShapeDtypeStruct((NC, NENT, H), jnp.float32),
        mesh=mesh,
        scratch_types=(
            [pltpu.VMEM_SHARED((NENT, H), jnp.float32)]
            + [pltpu.VMEM_SHARED((R, H), jnp.float32)]
            + [pltpu.VMEM((4, KE), jnp.int32)] * SRING
            + [pltpu.VMEM((KE, H), jnp.float32)] * RRING
            + [pltpu.SemaphoreType.DMA] * (SRING + 3 * RRING)
        ),
    )
    def sc_scatter(relval_hbm, rels_hbm, tails_hbm, heads_hbm, w_hbm,
                   out_hbm, acc, rvs, *bufs):
        edge_hbm = (rels_hbm, tails_hbm, heads_hbm, w_hbm)
        slabs = bufs[0:SRING]
        rows = bufs[SRING:SRING + RRING]
        ssem = bufs[SRING + RRING:2 * SRING + RRING]
        gsem = bufs[2 * SRING + RRING:2 * SRING + 2 * RRING]
        tsem = bufs[2 * SRING + 2 * RRING:2 * SRING + 3 * RRING]
        hsem = bufs[2 * SRING + 3 * RRING:2 * SRING + 4 * RRING]
        cid = lax.axis_index("c")
        sid = lax.axis_index("s")
        nbc = jnp.where(cid == 0, NB0, NB1)
        tbase = (cid * NS * NB0 + sid * nbc) * KE

        def zfill(r, carry):
            for c in range(H // LANES):
                rows[0][r, pl.ds(c * LANES, LANES)] = jnp.zeros(
                    (LANES,), jnp.float32)
            return carry

        lax.fori_loop(0, zch, zfill, 0)
        nch0 = zbase + jnp.where(sid < zextra, 1, 0)

        def zero_body(j, carry):
            c = sid + NS * j
            pltpu.sync_copy(rows[0].at[pl.ds(0, zch)],
                            acc.at[pl.ds(c * zch, zch)])
            return carry

        lax.fori_loop(0, nch0, zero_body, 0)

        nrc = rbase + jnp.where(sid < rextra, 1, 0)

        def rload_body(j, carry):
            c = sid + NS * j
            s = pl.ds(c * rch, rch)
            pltpu.sync_copy(relval_hbm.at[s], rvs.at[s])
            return carry

        lax.fori_loop(0, nrc, rload_body, 0)

        def slab_start(j, s):
            eoff = tbase + j * KE
            for a in range(4):
                pltpu.async_copy(edge_hbm[a].at[pl.ds(eoff, KE)],
                                 slabs[s].at[a], ssem[s])

        def slab_wait(j, s):
            eoff = tbase + j * KE
            for a in range(4):
                pltpu.make_async_copy(edge_hbm[a].at[pl.ds(eoff, KE)],
                                      slabs[s].at[a], ssem[s]).wait()

        def gather_start(s, r):
            pltpu.async_copy(rvs.at[slabs[s].at[0]], rows[r], gsem[r])

        def gather_wait(s, r):
            pltpu.make_async_copy(rvs.at[slabs[s].at[0]], rows[r],
                                  gsem[r]).wait()

        def scat_start(s, r):
            pltpu.async_copy(rows[r], acc.at[slabs[s].at[1]], tsem[r],
                             add=True)
            pltpu.async_copy(rows[r], acc.at[slabs[s].at[2]], hsem[r],
                             add=True)

        def scat_wait(s, r):
            pltpu.make_async_copy(rows[r], acc.at[slabs[s].at[1]],
                                  tsem[r]).wait()
            pltpu.make_async_copy(rows[r], acc.at[slabs[s].at[2]],
                                  hsem[r]).wait()

        def scale(s, r):
            sl, rw = slabs[s], rows[r]

            def grp_body(grp, c2):
                wvec = lax.bitcast_convert_type(
                    sl[3, pl.ds(grp * LANES, LANES)], jnp.float32)
                for i in range(LANES):
                    k = grp * LANES + i
                    wk = wvec[i]
                    for c in range(H // LANES):
                        cs = pl.ds(c * LANES, LANES)
                        rw[k, cs] = rw[k, cs] * wk
                return c2

            lax.fori_loop(0, KE // LANES, grp_body, 0)

        plsc.subcore_barrier()
        for s in range(4):
            slab_start(s, s)
        slab_wait(0, 0)
        gather_start(0, 0)

        def ring_body(g, carry):
            for k in range(SRING):
                j = SRING * g + k

                def step(jv):
                    scat_wait((k + 5) % SRING, (k + 1) % RRING)

                if k < 1:
                    @pl.when(g >= 1)
                    def _():
                        step(j)
                else:
                    step(j)
                slab_start(jnp.minimum(j + 4, nbc - 1), (k + 4) % SRING)
                slab_wait(jnp.minimum(j + 1, nbc - 1), (k + 1) % SRING)
                gather_start((k + 1) % SRING, (k + 1) % RRING)
                gather_wait(k % SRING, k % RRING)
                scale(k % SRING, k % RRING)
                scat_start(k % SRING, k % RRING)
            return carry

        lax.fori_loop(0, nbc // SRING, ring_body, 0)
        scat_wait(SRING - 1, (SRING - 1) % RRING)
        gather_wait(0, 0)
        for d in range(1, 4):
            slab_wait(nbc - 1, d % SRING)
        plsc.subcore_barrier()

        nch = cbase + jnp.where(sid < cextra, 1, 0)

        def out_body(j, carry):
            c = sid + NS * j
            s = pl.ds(c * chunk, chunk)
            pltpu.sync_copy(acc.at[s], out_hbm.at[cid, s])
            return carry

        lax.fori_loop(0, nch, out_body, 0)

    return sc_scatter


def kernel(local_entity, batch_heads, batch_rels, batch_tails, batch_ids,
           fact_ids, weight_list, weight_rel_list, rel_features, W, b):
    bsz, max_local_entity = local_entity.shape
    nent = bsz * max_local_entity
    R, H = rel_features.shape
    E = batch_rels.shape[0]

    rel_val = pl.pallas_call(
        _relval_body,
        out_shape=jax.ShapeDtypeStruct((R, H), jnp.float32),
    )(rel_features, W, b.reshape(1, H))

    nbsum = -(-E // (NS * KE))
    nb1 = max(2 * SRING, -(-(nbsum * 3) // (5 * SRING)) * SRING)
    nb0 = max(2 * SRING, -(-(nbsum - nb1) // SRING) * SRING)
    ep = NS * (nb0 + nb1) * KE - E

    def prep(x):
        return jnp.pad(x.astype(jnp.int32), (0, ep))

    part = _make_sc_scatter(nb0, nb1, nent, R, H)(
        rel_val,
        prep(batch_rels),
        prep(batch_tails),
        prep(batch_heads),
        prep(lax.bitcast_convert_type(
            weight_rel_list.astype(jnp.float32), jnp.int32)),
    )

    rows_blk = 2000
    out = pl.pallas_call(
        _addrelu_body,
        grid=(nent // rows_blk,),
        in_specs=[pl.BlockSpec((NC, rows_blk, H), lambda i: (0, i, 0))],
        out_specs=pl.BlockSpec((rows_blk, H), lambda i: (i, 0)),
        out_shape=jax.ShapeDtypeStruct((nent, H), jnp.float32),
    )(part)
    return out.reshape(bsz, max_local_entity, H)


# speedup vs baseline: 2.4610x; 1.1475x over previous
import functools

import jax
import jax.numpy as jnp
from jax import lax
from jax.experimental import pallas as pl
from jax.experimental.pallas import tpu as pltpu
from jax.experimental.pallas import tpu_sc as plsc

NC = 2
NS = 16
NT = NC * NS
KE = 112
LANES = 16
SRING = 6
RRING = 2


def _relval_body(rel_ref, w_ref, b_ref, out_ref):
    out_ref[...] = lax.dot_general(
        rel_ref[...], w_ref[...], (((1,), (1,)), ((), ())),
        preferred_element_type=jnp.float32) + b_ref[...]


def _addrelu_body(p_ref, o_ref):
    o_ref[...] = jnp.maximum(p_ref[0] + p_ref[1], 0.0)


@functools.lru_cache(maxsize=None)
def _make_sc_scatter(NB0, NB1, NENT, R, H):
    assert NB0 % SRING == 0 and NB0 >= 2 * SRING
    assert NB1 % SRING == 0 and NB1 >= 2 * SRING
    zch = 80
    nzch = NENT // zch
    zbase, zextra = nzch // NS, nzch % NS
    chunk = 200
    nchunk = NENT // chunk
    cbase, cextra = nchunk // NS, nchunk % NS
    rch = 40
    assert R % rch == 0
    nrch = R // rch
    rbase, rextra = nrch // NS, nrch % NS
    mesh = plsc.VectorSubcoreMesh(core_axis_name="c", subcore_axis_name="s")

    @functools.partial(
        pl.kernel,
        out_type=jax.ShapeDtypeStruct((NC, NENT, H), jnp.float32),
        mesh=mesh,
        scratch_types=(
            [pltpu.VMEM_SHARED((NENT, H), jnp.float32)]
            + [pltpu.VMEM_SHARED((R, H), jnp.float32)]
            + [pltpu.VMEM((4, KE), jnp.int32)] * SRING
            + [pltpu.VMEM((KE, H), jnp.float32)] * RRING
            + [pltpu.SemaphoreType.DMA] * (SRING + 3 * RRING)
        ),
    )
    def sc_scatter(relval_hbm, rels_hbm, tails_hbm, heads_hbm, w_hbm,
                   out_hbm, acc, rvs, *bufs):
        edge_hbm = (rels_hbm, tails_hbm, heads_hbm, w_hbm)
        slabs = bufs[0:SRING]
        rows = bufs[SRING:SRING + RRING]
        ssem = bufs[SRING + RRING:2 * SRING + RRING]
        gsem = bufs[2 * SRING + RRING:2 * SRING + 2 * RRING]
        tsem = bufs[2 * SRING + 2 * RRING:2 * SRING + 3 * RRING]
        hsem = bufs[2 * SRING + 3 * RRING:2 * SRING + 4 * RRING]
        cid = lax.axis_index("c")
        sid = lax.axis_index("s")
        nbc = jnp.where(cid == 0, NB0, NB1)
        tbase = (cid * NS * NB0 + sid * nbc) * KE

        def zfill(r, carry):
            for c in range(H // LANES):
                rows[0][r, pl.ds(c * LANES, LANES)] = jnp.zeros(
                    (LANES,), jnp.float32)
            return carry

        lax.fori_loop(0, zch, zfill, 0)
        nch0 = zbase + jnp.where(sid < zextra, 1, 0)

        def zero_body(j, carry):
            c = sid + NS * j
            pltpu.sync_copy(rows[0].at[pl.ds(0, zch)],
                            acc.at[pl.ds(c * zch, zch)])
            return carry

        lax.fori_loop(0, nch0, zero_body, 0)

        nrc = rbase + jnp.where(sid < rextra, 1, 0)

        def rload_body(j, carry):
            c = sid + NS * j
            s = pl.ds(c * rch, rch)
            pltpu.sync_copy(relval_hbm.at[s], rvs.at[s])
            return carry

        lax.fori_loop(0, nrc, rload_body, 0)

        def slab_start(j, s):
            eoff = tbase + j * KE
            for a in range(4):
                pltpu.async_copy(edge_hbm[a].at[pl.ds(eoff, KE)],
                                 slabs[s].at[a], ssem[s])

        def slab_wait(j, s):
            eoff = tbase + j * KE
            for a in range(4):
                pltpu.make_async_copy(edge_hbm[a].at[pl.ds(eoff, KE)],
                                      slabs[s].at[a], ssem[s]).wait()

        def gather_start(s, r):
            pltpu.async_copy(rvs.at[slabs[s].at[0]], rows[r], gsem[r])

        def gather_wait(s, r):
            pltpu.make_async_copy(rvs.at[slabs[s].at[0]], rows[r],
                                  gsem[r]).wait()

        def scat_start(s, r):
            pltpu.async_copy(rows[r], acc.at[slabs[s].at[1]], tsem[r],
                             add=True)
            pltpu.async_copy(rows[r], acc.at[slabs[s].at[2]], hsem[r],
                             add=True)

        def scat_wait(s, r):
            pltpu.make_async_copy(rows[r], acc.at[slabs[s].at[1]],
                                  tsem[r]).wait()
            pltpu.make_async_copy(rows[r], acc.at[slabs[s].at[2]],
                                  hsem[r]).wait()

        def scale(s, r):
            sl, rw = slabs[s], rows[r]

            def grp_body(grp, c2):
                wvec = lax.bitcast_convert_type(
                    sl[3, pl.ds(grp * LANES, LANES)], jnp.float32)
                for i in range(LANES):
                    k = grp * LANES + i
                    wk = wvec[i]
                    for c in range(H // LANES):
                        cs = pl.ds(c * LANES, LANES)
                        rw[k, cs] = rw[k, cs] * wk
                return c2

            lax.fori_loop(0, KE // LANES, grp_body, 0)

        plsc.subcore_barrier()
        for s in range(4):
            slab_start(s, s)
        slab_wait(0, 0)
        gather_start(0, 0)

        def ring_body(g, carry):
            for k in range(SRING):
                j = SRING * g + k

                def step(jv):
                    scat_wait((k + 5) % SRING, (k + 1) % RRING)

                if k < 1:
                    @pl.when(g >= 1)
                    def _():
                        step(j)
                else:
                    step(j)
                slab_start(jnp.minimum(j + 4, nbc - 1), (k + 4) % SRING)
                slab_wait(jnp.minimum(j + 1, nbc - 1), (k + 1) % SRING)
                gather_start((k + 1) % SRING, (k + 1) % RRING)
                gather_wait(k % SRING, k % RRING)
                scale(k % SRING, k % RRING)
                scat_start(k % SRING, k % RRING)
            return carry

        lax.fori_loop(0, nbc // SRING, ring_body, 0)
        scat_wait(SRING - 1, (SRING - 1) % RRING)
        gather_wait(0, 0)
        for d in range(1, 4):
            slab_wait(nbc - 1, d % SRING)
        plsc.subcore_barrier()

        nch = cbase + jnp.where(sid < cextra, 1, 0)

        def out_body(j, carry):
            c = sid + NS * j
            s = pl.ds(c * chunk, chunk)
            pltpu.sync_copy(acc.at[s], out_hbm.at[cid, s])
            return carry

        lax.fori_loop(0, nch, out_body, 0)

    return sc_scatter


def kernel(local_entity, batch_heads, batch_rels, batch_tails, batch_ids,
           fact_ids, weight_list, weight_rel_list, rel_features, W, b):
    bsz, max_local_entity = local_entity.shape
    nent = bsz * max_local_entity
    R, H = rel_features.shape
    E = batch_rels.shape[0]

    rel_val = pl.pallas_call(
        _relval_body,
        out_shape=jax.ShapeDtypeStruct((R, H), jnp.float32),
    )(rel_features, W, b.reshape(1, H))

    nbsum = -(-E // (NS * KE))
    nb1 = max(2 * SRING, -(-nbsum // (2 * SRING)) * SRING)
    nb0 = max(2 * SRING, -(-(nbsum - nb1) // SRING) * SRING)
    ep = NS * (nb0 + nb1) * KE - E

    def prep(x):
        return jnp.pad(x.astype(jnp.int32), (0, ep))

    part = _make_sc_scatter(nb0, nb1, nent, R, H)(
        rel_val,
        prep(batch_rels),
        prep(batch_tails),
        prep(batch_heads),
        prep(lax.bitcast_convert_type(
            weight_rel_list.astype(jnp.float32), jnp.int32)),
    )

    rows_blk = 2000
    out = pl.pallas_call(
        _addrelu_body,
        grid=(nent // rows_blk,),
        in_specs=[pl.BlockSpec((NC, rows_blk, H), lambda i: (0, i, 0))],
        out_specs=pl.BlockSpec((rows_blk, H), lambda i: (i, 0)),
        out_shape=jax.ShapeDtypeStruct((nent, H), jnp.float32),
    )(part)
    return out.reshape(bsz, max_local_entity, H)
